# Initial kernel scaffold; baseline (speedup 1.0000x reference)
import jax
import jax.numpy as jnp
from jax.experimental import pallas as pl


def kernel(x, edge_index, W, b, gamma, beta, alpha, W2, b2):
    raise NotImplementedError("write your pallas kernel here")


# speedup vs baseline: 7.5594x; 7.5594x over previous
import functools

import jax
import jax.numpy as jnp
from jax import lax
from jax.experimental import pallas as pl
from jax.experimental.pallas import tpu as pltpu
from jax.experimental.pallas import tpu_sc as plsc

N_NODES = 10000
D = 128
N_EDGES = 320000

NC = 2
NS = 16
NW = NC * NS
EDGES_PER_W = N_EDGES // NW
CHUNK = 80
N_CHUNKS = EDGES_PER_W // CHUNK
ROWS_PER_TILE = 640
N_PAD = ROWS_PER_TILE * NS


def _mm_body(x_ref, w_ref, b_ref, h_ref):
    h_ref[...] = (
        jnp.dot(x_ref[...], w_ref[...], preferred_element_type=jnp.float32)
        + b_ref[...]
    )


def _matmul(x, W, b2d):
    return pl.pallas_call(
        _mm_body,
        out_shape=jax.ShapeDtypeStruct((N_NODES, D), jnp.float32),
    )(x, W, b2d)


def _sc_agg(h, src_r, dst_r, zeros):
    mesh = plsc.VectorSubcoreMesh(core_axis_name="c", subcore_axis_name="s")

    @functools.partial(
        pl.kernel,
        mesh=mesh,
        out_type=jax.ShapeDtypeStruct((NC, N_PAD, D), jnp.float32),
        scratch_types=[
            pltpu.VMEM((N_CHUNKS, CHUNK), jnp.int32),
            pltpu.VMEM((N_CHUNKS, CHUNK), jnp.int32),
            pltpu.VMEM((CHUNK, D), jnp.float32),
            pltpu.VMEM_SHARED((N_PAD, D), jnp.float32),
            pltpu.SemaphoreType.DMA,
        ],
    )
    def k(h_hbm, src_hbm, dst_hbm, z_hbm, out_hbm, src_v, dst_v, rows_v, acc, sem):
        c = lax.axis_index("c")
        s = lax.axis_index("s")
        wid = c * NS + s

        pltpu.sync_copy(z_hbm, acc.at[pl.ds(s * ROWS_PER_TILE, ROWS_PER_TILE)])
        pltpu.sync_copy(src_hbm.at[wid], src_v)
        pltpu.sync_copy(dst_hbm.at[wid], dst_v)
        plsc.subcore_barrier()

        def body(i, carry):
            pltpu.async_copy(h_hbm.at[src_v.at[i]], rows_v, sem).wait()
            pltpu.sync_copy(rows_v, acc.at[dst_v.at[i]], add=True)
            return carry

        lax.fori_loop(0, N_CHUNKS, body, 0)

        plsc.subcore_barrier()
        pltpu.sync_copy(
            acc.at[pl.ds(s * ROWS_PER_TILE, ROWS_PER_TILE)],
            out_hbm.at[c, pl.ds(s * ROWS_PER_TILE, ROWS_PER_TILE)],
        )

    return k(h, src_r, dst_r, zeros)


def _finish_body(p_ref, g_ref, be_ref, al_ref, w2_ref, b2_ref, o_ref):
    agg = p_ref[0] + p_ref[1]
    n = float(N_NODES)
    mean = jnp.sum(agg, axis=0, keepdims=True) / n
    e2 = jnp.sum(agg * agg, axis=0, keepdims=True) / n
    var = e2 - mean * mean
    scale = g_ref[...] * lax.rsqrt(var + 1e-3)
    hn = (agg - mean) * scale + be_ref[...]
    act = jnp.where(hn > 0, hn, al_ref[...] * hn)
    row = lax.broadcasted_iota(jnp.int32, (N_PAD, 1), 0)
    act = jnp.where(row < N_NODES, act, 0.0)
    pooled = jnp.sum(act, axis=0, keepdims=True)
    o_ref[...] = jnp.sum(pooled * w2_ref[...], keepdims=True) + b2_ref[...]


def _finish(partials, gamma, beta, alpha, W2t, b2):
    return pl.pallas_call(
        _finish_body,
        out_shape=jax.ShapeDtypeStruct((1, 1), jnp.float32),
    )(partials, gamma, beta, alpha, W2t, b2)


def kernel(x, edge_index, W, b, gamma, beta, alpha, W2, b2):
    h = _matmul(x, W, b.reshape(1, D))
    src_r = edge_index[0].reshape(NW, N_CHUNKS, CHUNK)
    dst_r = edge_index[1].reshape(NW, N_CHUNKS, CHUNK)
    zeros = jnp.zeros((ROWS_PER_TILE, D), jnp.float32)
    partials = _sc_agg(h, src_r, dst_r, zeros)
    out = _finish(
        partials,
        gamma.reshape(1, D),
        beta.reshape(1, D),
        alpha.reshape(1, D),
        W2.reshape(1, D),
        b2.reshape(1, 1),
    )
    return out.reshape(1)
